# baseline (speedup 1.0000x reference)
import functools

import jax
import jax.numpy as jnp
from jax import lax
from jax.experimental import pallas as pl
from jax.experimental.pallas import tpu as pltpu
from jax.experimental.pallas import tpu_sc as plsc

B = 16384
D = 64
W = 128
L = 16
NC = 2
NS = 16
NW = NC * NS
RPW = B // NW
CH = 128
NCHUNK = RPW // CH


def _sc_body(e1_hbm, rel_hbm, e2_hbm, er_hbm, ei_hbm, rr_hbm, ri_hbm,
             out_hbm,
             e1_v, rel_v, e2_v, e1w_v, relw_v, e2w_v,
             pa_v, pr_v, pb_v,
             a_r, a_i, r_r, r_i, b_r, b_i,
             out_v, sem):
    wid = lax.axis_index("s") * NC + lax.axis_index("c")
    row0 = wid * RPW

    def chunk_body(c, carry):
        base = row0 + c * CH
        pltpu.sync_copy(e1_hbm.at[pl.ds(base, CH)], e1_v)
        pltpu.sync_copy(rel_hbm.at[pl.ds(base, CH)], rel_v)
        pltpu.sync_copy(e2_hbm.at[pl.ds(base, CH)], e2_v)

        def halve_body(i, carry2):
            sl = pl.ds(pl.multiple_of(i * L, L), L)
            e1c = e1_v[sl]
            rlc = rel_v[sl]
            e2c = e2_v[sl]
            e1w_v[sl] = lax.shift_right_logical(e1c, 1)
            relw_v[sl] = lax.shift_right_logical(rlc, 1)
            e2w_v[sl] = lax.shift_right_logical(e2c, 1)
            pa_v[sl] = (e1c & 1) * D
            pr_v[sl] = (rlc & 1) * D
            pb_v[sl] = (e2c & 1) * D
            return carry2

        lax.fori_loop(0, CH // L, halve_body, 0)

        cps = [
            pltpu.async_copy(er_hbm.at[e1w_v], a_r, sem),
            pltpu.async_copy(ei_hbm.at[e1w_v], a_i, sem),
            pltpu.async_copy(rr_hbm.at[relw_v], r_r, sem),
            pltpu.async_copy(ri_hbm.at[relw_v], r_i, sem),
            pltpu.async_copy(er_hbm.at[e2w_v], b_r, sem),
            pltpu.async_copy(ei_hbm.at[e2w_v], b_i, sem),
        ]
        for cp in cps:
            cp.wait()

        def group_body(g, carry2):
            sl = pl.ds(pl.multiple_of(g * L, L), L)
            rowv = g * L + lax.iota(jnp.int32, L)
            pav = pa_v[sl]
            prv = pr_v[sl]
            pbv = pb_v[sl]

            def d_body(d, acc):
                ca = pav + d
                cr = prv + d
                cb = pbv + d
                ar = plsc.load_gather(a_r, [rowv, ca])
                ai = plsc.load_gather(a_i, [rowv, ca])
                rr = plsc.load_gather(r_r, [rowv, cr])
                ri = plsc.load_gather(r_i, [rowv, cr])
                br = plsc.load_gather(b_r, [rowv, cb])
                bi = plsc.load_gather(b_i, [rowv, cb])
                return acc + br * (ar * rr - ai * ri) + bi * (ar * ri + ai * rr)

            acc = lax.fori_loop(0, D, d_body, jnp.zeros((L,), jnp.float32))
            res = 1.0 / (1.0 + jnp.exp(-acc))
            off = pl.multiple_of(c * CH + g * L, L)
            out_v[pl.ds(off, L)] = res
            return carry2

        lax.fori_loop(0, CH // L, group_body, 0)
        return carry

    lax.fori_loop(0, NCHUNK, chunk_body, 0)
    pltpu.sync_copy(out_v, out_hbm.at[pl.ds(row0, RPW)])


@jax.jit
def _scores(e1_idx, rel_idx, e2_idx, ent_real2, ent_img2, rel_real2, rel_img2):
    mesh = plsc.VectorSubcoreMesh(core_axis_name="c", subcore_axis_name="s")
    fn = pl.kernel(
        _sc_body,
        mesh=mesh,
        compiler_params=pltpu.CompilerParams(needs_layout_passes=False),
        out_type=jax.ShapeDtypeStruct((B,), jnp.float32),
        scratch_types=[
            pltpu.VMEM((CH,), jnp.int32),
            pltpu.VMEM((CH,), jnp.int32),
            pltpu.VMEM((CH,), jnp.int32),
            pltpu.VMEM((CH,), jnp.int32),
            pltpu.VMEM((CH,), jnp.int32),
            pltpu.VMEM((CH,), jnp.int32),
            pltpu.VMEM((CH,), jnp.int32),
            pltpu.VMEM((CH,), jnp.int32),
            pltpu.VMEM((CH,), jnp.int32),
            pltpu.VMEM((CH, W), jnp.float32),
            pltpu.VMEM((CH, W), jnp.float32),
            pltpu.VMEM((CH, W), jnp.float32),
            pltpu.VMEM((CH, W), jnp.float32),
            pltpu.VMEM((CH, W), jnp.float32),
            pltpu.VMEM((CH, W), jnp.float32),
            pltpu.VMEM((RPW,), jnp.float32),
            pltpu.SemaphoreType.DMA,
        ],
    )
    return fn(e1_idx, rel_idx, e2_idx, ent_real2, ent_img2, rel_real2, rel_img2)


def kernel(e1_idx, rel_idx, e2_idx, ent_real, ent_img, rel_real, rel_img):
    e1 = e1_idx.astype(jnp.int32)
    rel = rel_idx.astype(jnp.int32)
    e2 = e2_idx.astype(jnp.int32)
    ne, d = ent_real.shape
    nr = rel_real.shape[0]
    er2 = ent_real.reshape(ne // 2, 2 * d)
    ei2 = ent_img.reshape(ne // 2, 2 * d)
    rr2 = rel_real.reshape(nr // 2, 2 * d)
    ri2 = rel_img.reshape(nr // 2, 2 * d)
    out = _scores(e1, rel, e2, er2, ei2, rr2, ri2)
    return (out, jnp.float32(0.0))
